# Initial kernel scaffold; baseline (speedup 1.0000x reference)
import jax
import jax.numpy as jnp
from jax.experimental import pallas as pl


def kernel(feat, edge_index, is_po, global_feat, Ws1, bs1, Ws2, bs2, Wn1, bn1, Wn2, bn2, Wg1, bg1, Wg2, bg2, Wo1, bo1, Wo2, bo2):
    raise NotImplementedError("write your pallas kernel here")


# speedup vs baseline: 9.3504x; 9.3504x over previous
import functools
import jax
import jax.numpy as jnp
from jax import lax
from jax.experimental import pallas as pl
from jax.experimental.pallas import tpu as pltpu
from jax.experimental.pallas import tpu_sc as plsc

_N = 10000
_E = 320000
_F = 128
_H = 128

_NS = 16
_EPW = _E // _NS
_CH = 128
_NFULL = _EPW // _CH
_TAIL = _EPW - _NFULL * _CH
_RPW = 624
_RREM = _N - _NS * _RPW


def _leaky(x):
    return jnp.where(x >= 0, x, 0.1 * x)


def _stage_a_body(feat_ref, ws1_ref, bs1_ref, ws2_ref, bs2_ref, pq_ref):
    x = feat_ref[...]
    h1 = jnp.dot(x, ws1_ref[...], preferred_element_type=jnp.float32) + bs1_ref[...]
    h1 = _leaky(h1)
    h = jnp.dot(h1, ws2_ref[...], preferred_element_type=jnp.float32) + bs2_ref[...]
    c = jnp.max(h, axis=0, keepdims=True)
    p = jnp.exp(h - c)
    pq_ref[0:_N, :] = p
    pq_ref[_N:2 * _N, :] = p * h


_stage_a = pl.pallas_call(
    _stage_a_body,
    out_shape=jax.ShapeDtypeStruct((2 * _N, _F), jnp.float32),
)


_sc_mesh = plsc.VectorSubcoreMesh(core_axis_name="c", subcore_axis_name="s")


@functools.partial(
    pl.kernel,
    out_type=jax.ShapeDtypeStruct((2, _N, _F), jnp.float32),
    mesh=_sc_mesh,
    scratch_types=[
        pltpu.VMEM((_CH,), jnp.int32),
        pltpu.VMEM((1, _CH), jnp.int32),
        pltpu.VMEM((_CH, _F), jnp.float32),
        pltpu.VMEM((_TAIL,), jnp.int32),
        pltpu.VMEM((1, _TAIL), jnp.int32),
        pltpu.VMEM((_TAIL, _F), jnp.float32),
        pltpu.VMEM_SHARED((_N, _F), jnp.float32),
        pltpu.SemaphoreType.DMA,
    ],
)
def _stage_b(srcs_hbm, dst_hbm, table_hbm, zeros_hbm, out_hbm,
             sidx, didx, rows, sidx_t, didx_t, rows_t, acc, sem):
    c = lax.axis_index("c")
    s = lax.axis_index("s")
    ebase = s * _EPW
    rbase = s * _RPW

    pltpu.sync_copy(zeros_hbm.at[pl.ds(rbase, _RPW)], acc.at[pl.ds(rbase, _RPW)])

    @pl.when(s == _NS - 1)
    def _():
        pltpu.sync_copy(zeros_hbm.at[pl.ds(_NS * _RPW, _RREM)],
                        acc.at[pl.ds(_NS * _RPW, _RREM)])

    plsc.subcore_barrier()

    def chunk(j, carry):
        off = pl.multiple_of(ebase + j * _CH, _CH)
        pltpu.sync_copy(srcs_hbm.at[c, pl.ds(off, _CH)], sidx)
        pltpu.sync_copy(dst_hbm.at[pl.ds(off, _CH)], didx.at[0])
        pltpu.async_copy(table_hbm.at[sidx], rows, sem).wait()
        pltpu.sync_copy(rows, acc.at[didx.at[0]], add=True)
        return carry

    lax.fori_loop(0, _NFULL, chunk, 0)

    off = ebase + _NFULL * _CH
    pltpu.sync_copy(srcs_hbm.at[c, pl.ds(off, _TAIL)], sidx_t)
    pltpu.sync_copy(dst_hbm.at[pl.ds(off, _TAIL)], didx_t.at[0])
    pltpu.async_copy(table_hbm.at[sidx_t], rows_t, sem).wait()
    pltpu.sync_copy(rows_t, acc.at[didx_t.at[0]], add=True)

    plsc.subcore_barrier()
    pltpu.sync_copy(acc.at[pl.ds(rbase, _RPW)], out_hbm.at[c, pl.ds(rbase, _RPW)])

    @pl.when(s == _NS - 1)
    def _():
        pltpu.sync_copy(acc.at[pl.ds(_NS * _RPW, _RREM)],
                        out_hbm.at[c, pl.ds(_NS * _RPW, _RREM)])


def _stage_c_body(es_ref, em_ref, feat_ref, gf_ref, ispo_ref,
                  wn1a_ref, wn1b_ref, bn1_ref, wn2_ref, bn2_ref,
                  wg1_ref, bg1_ref, wg2_ref, bg2_ref,
                  wo1a_ref, wo1b_ref, bo1_ref, wo2_ref, bo2_ref,
                  out_ref):
    es = es_ref[...]
    em = em_ref[...]
    neigh = jnp.where(es > 0, em / es, 0.0)
    feat = feat_ref[...]
    h1 = (jnp.dot(neigh, wn1a_ref[...], preferred_element_type=jnp.float32)
          + jnp.dot(feat, wn1b_ref[...], preferred_element_type=jnp.float32)
          + bn1_ref[...])
    h1 = _leaky(h1)
    h = jnp.dot(h1, wn2_ref[...], preferred_element_type=jnp.float32) + bn2_ref[...]
    mask = ispo_ref[...] != 1
    h = jnp.where(mask, jnp.maximum(h, 0.0), h)
    g1 = _leaky(gf_ref[...] * wg1_ref[...] + bg1_ref[...])
    hg = jnp.dot(g1, wg2_ref[...], preferred_element_type=jnp.float32) + bg2_ref[...]
    o1 = (jnp.dot(h, wo1a_ref[...], preferred_element_type=jnp.float32)
          + jnp.dot(hg, wo1b_ref[...], preferred_element_type=jnp.float32)
          + bo1_ref[...])
    o1 = _leaky(o1)
    out_ref[...] = jnp.dot(o1, wo2_ref[...], preferred_element_type=jnp.float32) + bo2_ref[...]


_stage_c = pl.pallas_call(
    _stage_c_body,
    out_shape=jax.ShapeDtypeStruct((_N, 1), jnp.float32),
)


def kernel(feat, edge_index, is_po, global_feat,
           Ws1, bs1, Ws2, bs2, Wn1, bn1, Wn2, bn2,
           Wg1, bg1, Wg2, bg2, Wo1, bo1, Wo2, bo2):
    src = edge_index[0].astype(jnp.int32)
    dst = edge_index[1].astype(jnp.int32)
    srcs = jnp.stack([src, src + _N])

    pq = _stage_a(feat, Ws1, bs1.reshape(1, -1), Ws2, bs2.reshape(1, -1))

    zeros = jnp.zeros((_N, _F), jnp.float32)
    sums = _stage_b(srcs, dst, pq, zeros)
    es = sums[0]
    em = sums[1]

    out = _stage_c(
        es, em, feat, global_feat, is_po,
        Wn1[:_H], Wn1[_H:], bn1.reshape(1, -1), Wn2, bn2.reshape(1, -1),
        Wg1, bg1.reshape(1, -1), Wg2, bg2.reshape(1, -1),
        Wo1[:_H], Wo1[_H:], bo1.reshape(1, -1), Wo2, bo2.reshape(1, -1),
    )
    return out
